# baseline (speedup 1.0000x reference)
import functools

import jax
import jax.numpy as jnp
from jax import lax
from jax.experimental import pallas as pl
from jax.experimental.pallas import tpu as pltpu
from jax.experimental.pallas import tpu_sc as plsc

NC = 2
NS = 16
NW = NC * NS
NBUF = 8


def _mesh():
    return plsc.VectorSubcoreMesh(core_axis_name="c", subcore_axis_name="s")


_SC_PARAMS = pltpu.CompilerParams(use_tc_tiling_on_sc=False,
                                  needs_layout_passes=False)


@functools.lru_cache(maxsize=None)
def _make_count(n_pad: int, kch: int, chunk: int):
    rows = n_pad // NS
    epw = kch * chunk

    @functools.partial(
        pl.kernel,
        out_type=jax.ShapeDtypeStruct((NC * n_pad,), jnp.float32),
        mesh=_mesh(),
        compiler_params=_SC_PARAMS,
        scratch_types=[
            pltpu.VMEM((kch, chunk), jnp.int32),
            pltpu.VMEM((chunk,), jnp.float32),
            pltpu.VMEM_SHARED((n_pad,), jnp.float32),
            pltpu.SemaphoreType.DMA,
        ],
    )
    def count(ei_hbm, ones_hbm, out_hbm, dst_v, ones_v, acc, ssem):
        c = lax.axis_index("c")
        s = lax.axis_index("s")
        w = c * NS + s
        r0 = s * rows
        pltpu.sync_copy(ei_hbm.at[1, w], dst_v)
        pltpu.sync_copy(ones_hbm.at[pl.ds(0, chunk)], ones_v)
        pltpu.sync_copy(ones_hbm.at[pl.ds(r0, rows)], acc.at[pl.ds(r0, rows)])
        plsc.subcore_barrier()

        @pl.loop(0, kch)
        def _(j):
            pltpu.async_copy(ones_v, acc.at[dst_v.at[j]], ssem, add=True)

        @pl.loop(0, kch)
        def _(j):
            pltpu.make_async_copy(ones_v, acc.at[dst_v.at[j]], ssem).wait()

        plsc.subcore_barrier()
        pltpu.sync_copy(acc.at[pl.ds(r0, rows)],
                        out_hbm.at[pl.ds(c * n_pad + r0, rows)])

    return count


def _ring_loop(kch, src_v, dst_v, g_hbm, acc, bufs, gsems, ssems):
    lead = NBUF // 2
    for j in range(lead):
        pltpu.async_copy(g_hbm.at[src_v.at[j]], bufs[j], gsems[j])

    @pl.loop(0, kch, step=NBUF)
    def _(j0):
        for b in range(NBUF):
            j = j0 + b
            bl = (b + lead) % NBUF
            pltpu.make_async_copy(g_hbm.at[src_v.at[j]],
                                  bufs[b], gsems[b]).wait()
            pltpu.async_copy(bufs[b], acc.at[dst_v.at[j]], ssems[b], add=True)

            @pl.when(j >= lead)
            def _():
                pltpu.make_async_copy(bufs[bl], acc.at[dst_v.at[j - lead]],
                                      ssems[bl]).wait()

            @pl.when(j + lead < kch)
            def _():
                pltpu.async_copy(g_hbm.at[src_v.at[j + lead]],
                                 bufs[bl], gsems[bl])

    for k in range(lead):
        jj = kch - lead + k
        pltpu.make_async_copy(bufs[jj % NBUF], acc.at[dst_v.at[jj]],
                              ssems[jj % NBUF]).wait()


@functools.lru_cache(maxsize=None)
def _make_prop(n_pad: int, kch: int, chunk: int, f: int):
    rows = n_pad // NS
    epw = kch * chunk

    @functools.partial(
        pl.kernel,
        out_type=jax.ShapeDtypeStruct((NC * n_pad, f), jnp.float32),
        mesh=_mesh(),
        compiler_params=_SC_PARAMS,
        scratch_types=[
            pltpu.VMEM((kch, chunk), jnp.int32),
            pltpu.VMEM((kch, chunk), jnp.int32),
            [pltpu.VMEM((chunk, f), jnp.float32)] * NBUF,
            pltpu.VMEM_SHARED((n_pad, f), jnp.float32),
            [pltpu.SemaphoreType.DMA] * NBUF,
            [pltpu.SemaphoreType.DMA] * NBUF,
        ],
    )
    def prop(ei_hbm, g_hbm, out_hbm,
             src_v, dst_v, bufs, acc, gsems, ssems):
        c = lax.axis_index("c")
        s = lax.axis_index("s")
        w = c * NS + s
        r0 = s * rows
        pltpu.sync_copy(ei_hbm.at[0, w], src_v)
        pltpu.sync_copy(ei_hbm.at[1, w], dst_v)
        pltpu.sync_copy(g_hbm.at[pl.ds(r0, rows)], acc.at[pl.ds(r0, rows)])
        plsc.subcore_barrier()
        _ring_loop(kch, src_v, dst_v, g_hbm, acc, bufs, gsems, ssems)
        plsc.subcore_barrier()
        pltpu.sync_copy(acc.at[pl.ds(r0, rows)],
                        out_hbm.at[pl.ds(c * n_pad + r0, rows)])

    return prop


@functools.lru_cache(maxsize=None)
def _make_prop_final(n_pad: int, kch: int, chunk: int):
    rows = n_pad // NS

    @functools.partial(
        pl.kernel,
        out_type=jax.ShapeDtypeStruct((n_pad,), jnp.float32),
        mesh=_mesh(),
        compiler_params=_SC_PARAMS,
        scratch_types=[
            [pltpu.VMEM((kch, chunk), jnp.int32)] * 2,
            [pltpu.VMEM((kch, chunk), jnp.int32)] * 2,
            [pltpu.VMEM((chunk,), jnp.float32)] * NBUF,
            pltpu.VMEM((rows,), jnp.float32),
            pltpu.VMEM((rows,), jnp.float32),
            pltpu.VMEM((rows,), jnp.float32),
            pltpu.VMEM((rows,), jnp.float32),
            pltpu.VMEM((16,), jnp.float32),
            pltpu.VMEM_SHARED((n_pad,), jnp.float32),
            [pltpu.SemaphoreType.DMA] * NBUF,
            [pltpu.SemaphoreType.DMA] * NBUF,
        ],
    )
    def propf(ei_hbm, g_hbm, cnt_hbm, b3_hbm, out_hbm,
              src_v, dst_v, bufs, abuf, c0buf, c1buf, obuf, b3v, acc,
              gsems, ssems):
        c = lax.axis_index("c")
        s = lax.axis_index("s")
        r0 = s * rows
        for q in range(2):
            pltpu.sync_copy(ei_hbm.at[0, 2 * s + q], src_v[q])
            pltpu.sync_copy(ei_hbm.at[1, 2 * s + q], dst_v[q])
        pltpu.sync_copy(g_hbm.at[pl.ds(r0, rows)], acc.at[pl.ds(r0, rows)])
        plsc.subcore_barrier()
        for q in range(2):
            _ring_loop(kch, src_v[q], dst_v[q], g_hbm, acc, bufs, gsems,
                       ssems)
        plsc.subcore_barrier()

        @pl.when(c == 0)
        def _():
            pltpu.sync_copy(acc.at[pl.ds(r0, rows)], abuf)
            pltpu.sync_copy(cnt_hbm.at[pl.ds(r0, rows)], c0buf)
            pltpu.sync_copy(cnt_hbm.at[pl.ds(n_pad + r0, rows)], c1buf)
            pltpu.sync_copy(b3_hbm, b3v)
            bv = b3v[...]

            @pl.loop(0, rows // 16)
            def _(i):
                lo = i * 16
                deg = c0buf[pl.ds(lo, 16)] + c1buf[pl.ds(lo, 16)] - 1.0
                bits = plsc.bitcast(deg, jnp.int32)
                y = plsc.bitcast(0x5F3759DF - (bits >> 1), jnp.float32)
                for _ in range(3):
                    y = y * (1.5 - 0.5 * deg * y * y)
                obuf[pl.ds(lo, 16)] = y * abuf[pl.ds(lo, 16)] + bv

            pltpu.sync_copy(obuf, out_hbm.at[pl.ds(r0, rows)])

    return propf


def _dinv(c0, c1):
    return lax.rsqrt(c0 + c1 - 1.0)


def _t1_body(c0_ref, c1_ref, x_ref, w1_ref, g1_ref):
    dinv = _dinv(c0_ref[...], c1_ref[...])
    u1 = jnp.dot(x_ref[...], w1_ref[...], preferred_element_type=jnp.float32)
    g1_ref[...] = u1 * dinv


def _t2_body(c0_ref, c1_ref, pa_ref, pb_ref, g1_ref, b1_ref, g2_ref):
    dinv = _dinv(c0_ref[...], c1_ref[...])
    s = pa_ref[...] + pb_ref[...] - g1_ref[...]
    h1 = jnp.maximum(dinv * s + b1_ref[...], 0.0)
    g2_ref[...] = dinv * h1


def _t3_body(c0_ref, c1_ref, pa_ref, pb_ref, g2_ref, w2_ref, b2_ref, w3_ref,
             g3_ref):
    dinv = _dinv(c0_ref[...], c1_ref[...])
    ah1 = dinv * (pa_ref[...] + pb_ref[...] - g2_ref[...])
    h2 = jnp.maximum(
        jnp.dot(ah1, w2_ref[...], preferred_element_type=jnp.float32)
        + b2_ref[...], 0.0)
    z = jnp.sum(h2 * w3_ref[...], axis=1, keepdims=True)
    g3_ref[...] = z * dinv


BR = 512


def _tc_call(body, grid, in_specs, out_w, n_pad, args):
    return pl.pallas_call(
        body,
        grid=(grid,),
        in_specs=in_specs,
        out_specs=pl.BlockSpec((BR, out_w), lambda i: (i, 0)),
        out_shape=jax.ShapeDtypeStruct((n_pad, out_w), jnp.float32),
    )(*args)


def kernel(x, edge_index, W1, b1, W2, b2, W3, b3):
    n, d = x.shape
    e = edge_index.shape[1]
    f1 = W1.shape[1]
    f2 = W2.shape[1]

    n_pad = -(-n // BR) * BR
    gb = n_pad // BR

    chunk = None
    if e % NW == 0:
        epw = e // NW
        for ch in range(128, 0, -1):
            if epw % ch == 0 and (epw // ch) % NBUF == 0:
                chunk = ch
                break
    if chunk is not None:
        kch = e // (NW * chunk)
        ei = edge_index.reshape(2, NW, kch, chunk)
    else:
        chunk = 128
        kch = -(-(-(-e // (NW * chunk))) // NBUF) * NBUF
        e_pad = NW * kch * chunk
        spare = max(n_pad - n, 1)
        pad_dst = n + (jnp.arange(e_pad - e, dtype=jnp.int32) % spare)
        ei = jnp.stack([
            jnp.concatenate([edge_index[0],
                             jnp.zeros((e_pad - e,), jnp.int32)]),
            jnp.concatenate([edge_index[1], pad_dst]),
        ]).reshape(2, NW, kch, chunk)

    ones = jnp.ones((n_pad,), jnp.float32)

    cnt = _make_count(n_pad, kch, chunk)(ei, ones)
    cnt2 = cnt.reshape(NC * n_pad, 1)

    def lo(w):
        return pl.BlockSpec((BR, w), lambda i: (i, 0))

    def hi(w):
        return pl.BlockSpec((BR, w), lambda i: (i + gb, 0))

    def full(shape):
        return pl.BlockSpec(shape, lambda i: (0, 0))

    cs = (lo(1), hi(1))

    g1 = _tc_call(
        _t1_body, gb,
        [*cs, lo(d), full((d, f1))],
        f1, n_pad, (cnt2, cnt2, x, W1))

    p1 = _make_prop(n_pad, kch, chunk, f1)(ei, g1)

    g2 = _tc_call(
        _t2_body, gb,
        [*cs, lo(f1), hi(f1), lo(f1), full((1, f1))],
        f1, n_pad, (cnt2, cnt2, p1, p1, g1, b1.reshape(1, f1)))

    p2 = _make_prop(n_pad, kch, chunk, f1)(ei, g2)

    g3 = _tc_call(
        _t3_body, gb,
        [*cs, lo(f1), hi(f1), lo(f1), full((f1, f2)), full((1, f2)),
         full((1, f2))],
        1, n_pad, (cnt2, cnt2, p2, p2, g2, W2, b2.reshape(1, f2),
                   W3.reshape(1, f2)))

    b3v = jnp.broadcast_to(b3.astype(jnp.float32).reshape(1), (16,))
    out = _make_prop_final(n_pad, kch, chunk)(
        ei, g3.reshape(n_pad), cnt, b3v)

    return out[:n]


# speedup vs baseline: 1.0442x; 1.0442x over previous
import functools

import jax
import jax.numpy as jnp
from jax import lax
from jax.experimental import pallas as pl
from jax.experimental.pallas import tpu as pltpu
from jax.experimental.pallas import tpu_sc as plsc

NC = 2
NS = 16
NW = NC * NS
NBUF = 8


def _mesh():
    return plsc.VectorSubcoreMesh(core_axis_name="c", subcore_axis_name="s")


_SC_PARAMS = pltpu.CompilerParams(use_tc_tiling_on_sc=False,
                                  needs_layout_passes=False)


@functools.lru_cache(maxsize=None)
def _make_count(n_pad: int, kch: int, chunk: int):
    rows = n_pad // NS
    epw = kch * chunk

    @functools.partial(
        pl.kernel,
        out_type=jax.ShapeDtypeStruct((NC * n_pad,), jnp.float32),
        mesh=_mesh(),
        compiler_params=_SC_PARAMS,
        scratch_types=[
            pltpu.VMEM((kch, chunk), jnp.int32),
            pltpu.VMEM((chunk,), jnp.float32),
            pltpu.VMEM_SHARED((n_pad,), jnp.float32),
            pltpu.SemaphoreType.DMA,
        ],
    )
    def count(dst_hbm, ones_hbm, out_hbm, dst_v, ones_v, acc, ssem):
        c = lax.axis_index("c")
        s = lax.axis_index("s")
        w = c * NS + s
        r0 = s * rows
        pltpu.sync_copy(dst_hbm.at[w], dst_v)
        pltpu.sync_copy(ones_hbm.at[pl.ds(0, chunk)], ones_v)
        pltpu.sync_copy(ones_hbm.at[pl.ds(r0, rows)], acc.at[pl.ds(r0, rows)])
        plsc.subcore_barrier()

        @pl.loop(0, kch)
        def _(j):
            pltpu.async_copy(ones_v, acc.at[dst_v.at[j]], ssem, add=True)

        @pl.loop(0, kch)
        def _(j):
            pltpu.make_async_copy(ones_v, acc.at[dst_v.at[j]], ssem).wait()

        plsc.subcore_barrier()
        pltpu.sync_copy(acc.at[pl.ds(r0, rows)],
                        out_hbm.at[pl.ds(c * n_pad + r0, rows)])

    return count


def _ring_loop(kch, src_v, dst_v, g_hbm, acc, bufs, gsems, ssems):
    lead = NBUF // 2
    for j in range(lead):
        pltpu.async_copy(g_hbm.at[src_v.at[j]], bufs[j], gsems[j])

    @pl.loop(0, kch, step=NBUF)
    def _(j0):
        for b in range(NBUF):
            j = j0 + b
            bl = (b + lead) % NBUF
            pltpu.make_async_copy(g_hbm.at[src_v.at[j]],
                                  bufs[b], gsems[b]).wait()
            pltpu.async_copy(bufs[b], acc.at[dst_v.at[j]], ssems[b], add=True)

            @pl.when(j >= lead)
            def _():
                pltpu.make_async_copy(bufs[bl], acc.at[dst_v.at[j - lead]],
                                      ssems[bl]).wait()

            @pl.when(j + lead < kch)
            def _():
                pltpu.async_copy(g_hbm.at[src_v.at[j + lead]],
                                 bufs[bl], gsems[bl])

    for k in range(lead):
        jj = kch - lead + k
        pltpu.make_async_copy(bufs[jj % NBUF], acc.at[dst_v.at[jj]],
                              ssems[jj % NBUF]).wait()


@functools.lru_cache(maxsize=None)
def _make_prop(n_pad: int, kch: int, chunk: int, f: int):
    rows = n_pad // NS
    epw = kch * chunk

    @functools.partial(
        pl.kernel,
        out_type=jax.ShapeDtypeStruct((NC * n_pad, f), jnp.float32),
        mesh=_mesh(),
        compiler_params=_SC_PARAMS,
        scratch_types=[
            pltpu.VMEM((kch, chunk), jnp.int32),
            pltpu.VMEM((kch, chunk), jnp.int32),
            [pltpu.VMEM((chunk, f), jnp.float32)] * NBUF,
            pltpu.VMEM_SHARED((n_pad, f), jnp.float32),
            [pltpu.SemaphoreType.DMA] * NBUF,
            [pltpu.SemaphoreType.DMA] * NBUF,
        ],
    )
    def prop(src_hbm, dst_hbm, g_hbm, out_hbm,
             src_v, dst_v, bufs, acc, gsems, ssems):
        c = lax.axis_index("c")
        s = lax.axis_index("s")
        w = c * NS + s
        r0 = s * rows
        pltpu.sync_copy(src_hbm.at[w], src_v)
        pltpu.sync_copy(dst_hbm.at[w], dst_v)
        pltpu.sync_copy(g_hbm.at[pl.ds(r0, rows)], acc.at[pl.ds(r0, rows)])
        plsc.subcore_barrier()
        _ring_loop(kch, src_v, dst_v, g_hbm, acc, bufs, gsems, ssems)
        plsc.subcore_barrier()
        pltpu.sync_copy(acc.at[pl.ds(r0, rows)],
                        out_hbm.at[pl.ds(c * n_pad + r0, rows)])

    return prop


@functools.lru_cache(maxsize=None)
def _make_prop_final(n_pad: int, kch: int, chunk: int):
    rows = n_pad // NS

    @functools.partial(
        pl.kernel,
        out_type=jax.ShapeDtypeStruct((n_pad,), jnp.float32),
        mesh=_mesh(),
        compiler_params=_SC_PARAMS,
        scratch_types=[
            [pltpu.VMEM((kch, chunk), jnp.int32)] * 2,
            [pltpu.VMEM((kch, chunk), jnp.int32)] * 2,
            [pltpu.VMEM((chunk,), jnp.float32)] * NBUF,
            pltpu.VMEM((rows,), jnp.float32),
            pltpu.VMEM((rows,), jnp.float32),
            pltpu.VMEM((rows,), jnp.float32),
            pltpu.VMEM((rows,), jnp.float32),
            pltpu.VMEM((16,), jnp.float32),
            pltpu.VMEM_SHARED((n_pad,), jnp.float32),
            [pltpu.SemaphoreType.DMA] * NBUF,
            [pltpu.SemaphoreType.DMA] * NBUF,
        ],
    )
    def propf(src_hbm, dst_hbm, g_hbm, cnt_hbm, b3_hbm, out_hbm,
              src_v, dst_v, bufs, abuf, c0buf, c1buf, obuf, b3v, acc,
              gsems, ssems):
        c = lax.axis_index("c")
        s = lax.axis_index("s")
        r0 = s * rows
        for q in range(2):
            pltpu.sync_copy(src_hbm.at[2 * s + q], src_v[q])
            pltpu.sync_copy(dst_hbm.at[2 * s + q], dst_v[q])
        pltpu.sync_copy(g_hbm.at[pl.ds(r0, rows)], acc.at[pl.ds(r0, rows)])
        plsc.subcore_barrier()
        for q in range(2):
            _ring_loop(kch, src_v[q], dst_v[q], g_hbm, acc, bufs, gsems,
                       ssems)
        plsc.subcore_barrier()

        @pl.when(c == 0)
        def _():
            pltpu.sync_copy(acc.at[pl.ds(r0, rows)], abuf)
            pltpu.sync_copy(cnt_hbm.at[pl.ds(r0, rows)], c0buf)
            pltpu.sync_copy(cnt_hbm.at[pl.ds(n_pad + r0, rows)], c1buf)
            pltpu.sync_copy(b3_hbm, b3v)
            bv = b3v[...]

            @pl.loop(0, rows // 16)
            def _(i):
                lo = i * 16
                deg = c0buf[pl.ds(lo, 16)] + c1buf[pl.ds(lo, 16)] - 1.0
                bits = plsc.bitcast(deg, jnp.int32)
                y = plsc.bitcast(0x5F3759DF - (bits >> 1), jnp.float32)
                for _ in range(3):
                    y = y * (1.5 - 0.5 * deg * y * y)
                obuf[pl.ds(lo, 16)] = y * abuf[pl.ds(lo, 16)] + bv

            pltpu.sync_copy(obuf, out_hbm.at[pl.ds(r0, rows)])

    return propf


def _dinv(c0, c1):
    return lax.rsqrt(c0 + c1 - 1.0)


def _t1_body(c0_ref, c1_ref, x_ref, w1_ref, g1_ref):
    dinv = _dinv(c0_ref[...], c1_ref[...])
    u1 = jnp.dot(x_ref[...], w1_ref[...], preferred_element_type=jnp.float32)
    g1_ref[...] = u1 * dinv


def _t2_body(c0_ref, c1_ref, pa_ref, pb_ref, g1_ref, b1_ref, g2_ref):
    dinv = _dinv(c0_ref[...], c1_ref[...])
    s = pa_ref[...] + pb_ref[...] - g1_ref[...]
    h1 = jnp.maximum(dinv * s + b1_ref[...], 0.0)
    g2_ref[...] = dinv * h1


def _t3_body(c0_ref, c1_ref, pa_ref, pb_ref, g2_ref, w2_ref, b2_ref, w3_ref,
             g3_ref):
    dinv = _dinv(c0_ref[...], c1_ref[...])
    ah1 = dinv * (pa_ref[...] + pb_ref[...] - g2_ref[...])
    h2 = jnp.maximum(
        jnp.dot(ah1, w2_ref[...], preferred_element_type=jnp.float32)
        + b2_ref[...], 0.0)
    z = jnp.sum(h2 * w3_ref[...], axis=1, keepdims=True)
    g3_ref[...] = z * dinv


BR = 512


def _tc_call(body, grid, in_specs, out_w, n_pad, args):
    return pl.pallas_call(
        body,
        grid=(grid,),
        in_specs=in_specs,
        out_specs=pl.BlockSpec((BR, out_w), lambda i: (i, 0)),
        out_shape=jax.ShapeDtypeStruct((n_pad, out_w), jnp.float32),
    )(*args)


def kernel(x, edge_index, W1, b1, W2, b2, W3, b3):
    n, d = x.shape
    e = edge_index.shape[1]
    f1 = W1.shape[1]
    f2 = W2.shape[1]

    n_pad = -(-n // BR) * BR
    gb = n_pad // BR

    chunk = None
    if e % NW == 0:
        epw = e // NW
        for ch in range(128, 0, -1):
            if epw % ch == 0 and (epw // ch) % NBUF == 0:
                chunk = ch
                break
    if chunk is not None:
        kch = e // (NW * chunk)
        src = edge_index[0].reshape(NW, kch, chunk)
        dst = edge_index[1].reshape(NW, kch, chunk)
    else:
        chunk = 128
        kch = -(-(-(-e // (NW * chunk))) // NBUF) * NBUF
        e_pad = NW * kch * chunk
        spare = max(n_pad - n, 1)
        pad_dst = n + (jnp.arange(e_pad - e, dtype=jnp.int32) % spare)
        src = jnp.concatenate(
            [edge_index[0], jnp.zeros((e_pad - e,), jnp.int32)]).reshape(
                NW, kch, chunk)
        dst = jnp.concatenate([edge_index[1], pad_dst]).reshape(
            NW, kch, chunk)

    ones = jnp.ones((n_pad,), jnp.float32)

    cnt = _make_count(n_pad, kch, chunk)(dst, ones)
    cnt2 = cnt.reshape(NC * n_pad, 1)

    def lo(w):
        return pl.BlockSpec((BR, w), lambda i: (i, 0))

    def hi(w):
        return pl.BlockSpec((BR, w), lambda i: (i + gb, 0))

    def full(shape):
        return pl.BlockSpec(shape, lambda i: (0, 0))

    cs = (lo(1), hi(1))

    g1 = _tc_call(
        _t1_body, gb,
        [*cs, lo(d), full((d, f1))],
        f1, n_pad, (cnt2, cnt2, x, W1))

    p1 = _make_prop(n_pad, kch, chunk, f1)(src, dst, g1)

    g2 = _tc_call(
        _t2_body, gb,
        [*cs, lo(f1), hi(f1), lo(f1), full((1, f1))],
        f1, n_pad, (cnt2, cnt2, p1, p1, g1, b1.reshape(1, f1)))

    p2 = _make_prop(n_pad, kch, chunk, f1)(src, dst, g2)

    g3 = _tc_call(
        _t3_body, gb,
        [*cs, lo(f1), hi(f1), lo(f1), full((f1, f2)), full((1, f2)),
         full((1, f2))],
        1, n_pad, (cnt2, cnt2, p2, p2, g2, W2, b2.reshape(1, f2),
                   W3.reshape(1, f2)))

    b3v = jnp.broadcast_to(b3.astype(jnp.float32).reshape(1), (16,))
    out = _make_prop_final(n_pad, kch, chunk)(
        src, dst, g3.reshape(n_pad), cnt, b3v)

    return out[:n]


# speedup vs baseline: 1.1403x; 1.0920x over previous
import functools

import jax
import jax.numpy as jnp
from jax import lax
from jax.experimental import pallas as pl
from jax.experimental.pallas import tpu as pltpu
from jax.experimental.pallas import tpu_sc as plsc

NC = 2
NS = 16
NW = NC * NS
NBUF = 8


def _mesh():
    return plsc.VectorSubcoreMesh(core_axis_name="c", subcore_axis_name="s")


_SC_PARAMS = pltpu.CompilerParams(use_tc_tiling_on_sc=False,
                                  needs_layout_passes=False)


@functools.lru_cache(maxsize=None)
def _make_count(n_pad: int, kch: int, chunk: int):
    rows = n_pad // NS
    epw = kch * chunk

    @functools.partial(
        pl.kernel,
        out_type=jax.ShapeDtypeStruct((NC * n_pad,), jnp.float32),
        mesh=_mesh(),
        compiler_params=_SC_PARAMS,
        scratch_types=[
            pltpu.VMEM((kch, chunk), jnp.int32),
            pltpu.VMEM((chunk,), jnp.float32),
            pltpu.VMEM_SHARED((n_pad,), jnp.float32),
            pltpu.SemaphoreType.DMA,
        ],
    )
    def count(dst_hbm, ones_hbm, out_hbm, dst_v, ones_v, acc, ssem):
        c = lax.axis_index("c")
        s = lax.axis_index("s")
        w = c * NS + s
        r0 = s * rows
        pltpu.sync_copy(dst_hbm.at[w], dst_v)
        pltpu.sync_copy(ones_hbm.at[pl.ds(0, chunk)], ones_v)
        pltpu.sync_copy(ones_hbm.at[pl.ds(r0, rows)], acc.at[pl.ds(r0, rows)])
        plsc.subcore_barrier()

        @pl.loop(0, kch)
        def _(j):
            pltpu.async_copy(ones_v, acc.at[dst_v.at[j]], ssem, add=True)

        @pl.loop(0, kch)
        def _(j):
            pltpu.make_async_copy(ones_v, acc.at[dst_v.at[j]], ssem).wait()

        plsc.subcore_barrier()
        pltpu.sync_copy(acc.at[pl.ds(r0, rows)],
                        out_hbm.at[pl.ds(c * n_pad + r0, rows)])

    return count


def _ring_loop(kch, src_v, dst_v, g_hbm, acc, bufs, gsems, ssems):
    lead = NBUF // 2
    for j in range(lead):
        pltpu.async_copy(g_hbm.at[src_v.at[j]], bufs[j], gsems[j])

    @pl.loop(0, kch, step=NBUF)
    def _(j0):
        for b in range(NBUF):
            j = j0 + b
            bl = (b + lead) % NBUF
            pltpu.make_async_copy(g_hbm.at[src_v.at[j]],
                                  bufs[b], gsems[b]).wait()
            pltpu.async_copy(bufs[b], acc.at[dst_v.at[j]], ssems[b], add=True)

            @pl.when(j >= lead)
            def _():
                pltpu.make_async_copy(bufs[bl], acc.at[dst_v.at[j - lead]],
                                      ssems[bl]).wait()

            @pl.when(j + lead < kch)
            def _():
                pltpu.async_copy(g_hbm.at[src_v.at[j + lead]],
                                 bufs[bl], gsems[bl])

    for k in range(lead):
        jj = kch - lead + k
        pltpu.make_async_copy(bufs[jj % NBUF], acc.at[dst_v.at[jj]],
                              ssems[jj % NBUF]).wait()


@functools.lru_cache(maxsize=None)
def _make_prop(n_pad: int, kch: int, chunk: int, f: int):
    rows = n_pad // NS
    epw = kch * chunk

    @functools.partial(
        pl.kernel,
        out_type=jax.ShapeDtypeStruct((NC * n_pad, f), jnp.float32),
        mesh=_mesh(),
        compiler_params=_SC_PARAMS,
        scratch_types=[
            pltpu.VMEM((kch, chunk), jnp.int32),
            pltpu.VMEM((kch, chunk), jnp.int32),
            [pltpu.VMEM((chunk, f), jnp.float32)] * NBUF,
            pltpu.VMEM_SHARED((n_pad, f), jnp.float32),
            [pltpu.SemaphoreType.DMA] * NBUF,
            [pltpu.SemaphoreType.DMA] * NBUF,
        ],
    )
    def prop(src_hbm, dst_hbm, g_hbm, out_hbm,
             src_v, dst_v, bufs, acc, gsems, ssems):
        c = lax.axis_index("c")
        s = lax.axis_index("s")
        w = c * NS + s
        r0 = s * rows
        pltpu.sync_copy(src_hbm.at[w], src_v)
        pltpu.sync_copy(dst_hbm.at[w], dst_v)
        pltpu.sync_copy(g_hbm.at[pl.ds(r0, rows)], acc.at[pl.ds(r0, rows)])
        plsc.subcore_barrier()
        _ring_loop(kch, src_v, dst_v, g_hbm, acc, bufs, gsems, ssems)
        plsc.subcore_barrier()
        pltpu.sync_copy(acc.at[pl.ds(r0, rows)],
                        out_hbm.at[pl.ds(c * n_pad + r0, rows)])

    return prop


@functools.lru_cache(maxsize=None)
def _make_prop1(n_pad: int, kch: int, chunk: int):
    rows = n_pad // NS

    @functools.partial(
        pl.kernel,
        out_type=jax.ShapeDtypeStruct((NC * n_pad,), jnp.float32),
        mesh=_mesh(),
        compiler_params=_SC_PARAMS,
        scratch_types=[
            pltpu.VMEM((kch, chunk), jnp.int32),
            pltpu.VMEM((kch, chunk), jnp.int32),
            [pltpu.VMEM((chunk,), jnp.float32)] * NBUF,
            pltpu.VMEM_SHARED((n_pad,), jnp.float32),
            [pltpu.SemaphoreType.DMA] * NBUF,
            [pltpu.SemaphoreType.DMA] * NBUF,
        ],
    )
    def prop1(src_hbm, dst_hbm, g_hbm, out_hbm,
              src_v, dst_v, bufs, acc, gsems, ssems):
        c = lax.axis_index("c")
        s = lax.axis_index("s")
        w = c * NS + s
        r0 = s * rows
        pltpu.sync_copy(src_hbm.at[w], src_v)
        pltpu.sync_copy(dst_hbm.at[w], dst_v)
        pltpu.sync_copy(g_hbm.at[pl.ds(r0, rows)], acc.at[pl.ds(r0, rows)])
        plsc.subcore_barrier()
        _ring_loop(kch, src_v, dst_v, g_hbm, acc, bufs, gsems, ssems)
        plsc.subcore_barrier()
        pltpu.sync_copy(acc.at[pl.ds(r0, rows)],
                        out_hbm.at[pl.ds(c * n_pad + r0, rows)])

    return prop1


@functools.lru_cache(maxsize=None)
def _make_final(n_pad: int):
    rpw = n_pad // NW

    @functools.partial(
        pl.kernel,
        out_type=jax.ShapeDtypeStruct((n_pad,), jnp.float32),
        mesh=_mesh(),
        compiler_params=_SC_PARAMS,
        scratch_types=[
            pltpu.VMEM((rpw,), jnp.float32),
            pltpu.VMEM((rpw,), jnp.float32),
            pltpu.VMEM((rpw,), jnp.float32),
            pltpu.VMEM((rpw,), jnp.float32),
            pltpu.VMEM((rpw,), jnp.float32),
            pltpu.VMEM((16,), jnp.float32),
        ],
    )
    def fin(p_hbm, cnt_hbm, g_hbm, b3_hbm, out_hbm,
            pa, pb, c0buf, c1buf, obuf, b3v):
        c = lax.axis_index("c")
        s = lax.axis_index("s")
        r0 = (c * NS + s) * rpw
        pltpu.sync_copy(p_hbm.at[pl.ds(r0, rpw)], pa)
        pltpu.sync_copy(p_hbm.at[pl.ds(n_pad + r0, rpw)], pb)
        pltpu.sync_copy(cnt_hbm.at[pl.ds(r0, rpw)], c0buf)
        pltpu.sync_copy(cnt_hbm.at[pl.ds(n_pad + r0, rpw)], c1buf)
        pltpu.sync_copy(g_hbm.at[pl.ds(r0, rpw)], obuf)
        pltpu.sync_copy(b3_hbm, b3v)
        bv = b3v[...]

        @pl.loop(0, rpw // 16)
        def _(i):
            lo = i * 16
            deg = c0buf[pl.ds(lo, 16)] + c1buf[pl.ds(lo, 16)] - 1.0
            bits = plsc.bitcast(deg, jnp.int32)
            y = plsc.bitcast(0x5F3759DF - (bits >> 1), jnp.float32)
            for _ in range(4):
                y = y * (1.5 - 0.5 * deg * y * y)
            acc = pa[pl.ds(lo, 16)] + pb[pl.ds(lo, 16)] - obuf[pl.ds(lo, 16)]
            obuf[pl.ds(lo, 16)] = y * acc + bv

        pltpu.sync_copy(obuf, out_hbm.at[pl.ds(r0, rpw)])

    return fin


def _dinv(c0, c1):
    return lax.rsqrt(c0 + c1 - 1.0)


def _t1_body(c0_ref, c1_ref, x_ref, w1_ref, g1_ref):
    dinv = _dinv(c0_ref[...], c1_ref[...])
    u1 = jnp.dot(x_ref[...], w1_ref[...], preferred_element_type=jnp.float32)
    g1_ref[...] = u1 * dinv


def _t2_body(c0_ref, c1_ref, pa_ref, pb_ref, g1_ref, b1_ref, g2_ref):
    dinv = _dinv(c0_ref[...], c1_ref[...])
    s = pa_ref[...] + pb_ref[...] - g1_ref[...]
    h1 = jnp.maximum(dinv * s + b1_ref[...], 0.0)
    g2_ref[...] = dinv * h1


def _t3_body(c0_ref, c1_ref, pa_ref, pb_ref, g2_ref, w2_ref, b2_ref, w3_ref,
             g3_ref):
    dinv = _dinv(c0_ref[...], c1_ref[...])
    ah1 = dinv * (pa_ref[...] + pb_ref[...] - g2_ref[...])
    h2 = jnp.maximum(
        jnp.dot(ah1, w2_ref[...], preferred_element_type=jnp.float32)
        + b2_ref[...], 0.0)
    z = jnp.sum(h2 * w3_ref[...], axis=1, keepdims=True)
    g3_ref[...] = z * dinv


BR = 512


def _tc_call(body, grid, in_specs, out_w, n_pad, args):
    return pl.pallas_call(
        body,
        grid=(grid,),
        in_specs=in_specs,
        out_specs=pl.BlockSpec((BR, out_w), lambda i: (i, 0)),
        out_shape=jax.ShapeDtypeStruct((n_pad, out_w), jnp.float32),
    )(*args)


def kernel(x, edge_index, W1, b1, W2, b2, W3, b3):
    n, d = x.shape
    e = edge_index.shape[1]
    f1 = W1.shape[1]
    f2 = W2.shape[1]

    n_pad = -(-n // BR) * BR
    gb = n_pad // BR

    chunk = None
    if e % NW == 0:
        epw = e // NW
        for ch in range(128, 0, -1):
            if epw % ch == 0 and (epw // ch) % NBUF == 0:
                chunk = ch
                break
    if chunk is not None:
        kch = e // (NW * chunk)
        src = edge_index[0].reshape(NW, kch, chunk)
        dst = edge_index[1].reshape(NW, kch, chunk)
    else:
        chunk = 128
        kch = -(-(-(-e // (NW * chunk))) // NBUF) * NBUF
        e_pad = NW * kch * chunk
        spare = max(n_pad - n, 1)
        pad_dst = n + (jnp.arange(e_pad - e, dtype=jnp.int32) % spare)
        src = jnp.concatenate(
            [edge_index[0], jnp.zeros((e_pad - e,), jnp.int32)]).reshape(
                NW, kch, chunk)
        dst = jnp.concatenate([edge_index[1], pad_dst]).reshape(
            NW, kch, chunk)

    ones = jnp.ones((n_pad,), jnp.float32)

    cnt = _make_count(n_pad, kch, chunk)(dst, ones)
    cnt2 = cnt.reshape(NC * n_pad, 1)

    def lo(w):
        return pl.BlockSpec((BR, w), lambda i: (i, 0))

    def hi(w):
        return pl.BlockSpec((BR, w), lambda i: (i + gb, 0))

    def full(shape):
        return pl.BlockSpec(shape, lambda i: (0, 0))

    cs = (lo(1), hi(1))

    g1 = _tc_call(
        _t1_body, gb,
        [*cs, lo(d), full((d, f1))],
        f1, n_pad, (cnt2, cnt2, x, W1))

    p1 = _make_prop(n_pad, kch, chunk, f1)(src, dst, g1)

    g2 = _tc_call(
        _t2_body, gb,
        [*cs, lo(f1), hi(f1), lo(f1), full((1, f1))],
        f1, n_pad, (cnt2, cnt2, p1, p1, g1, b1.reshape(1, f1)))

    p2 = _make_prop(n_pad, kch, chunk, f1)(src, dst, g2)

    g3 = _tc_call(
        _t3_body, gb,
        [*cs, lo(f1), hi(f1), lo(f1), full((f1, f2)), full((1, f2)),
         full((1, f2))],
        1, n_pad, (cnt2, cnt2, p2, p2, g2, W2, b2.reshape(1, f2),
                   W3.reshape(1, f2)))

    b3v = jnp.broadcast_to(b3.astype(jnp.float32).reshape(1), (16,))
    g3f = g3.reshape(n_pad)
    p3 = _make_prop1(n_pad, kch, chunk)(src, dst, g3f)
    out = _make_final(n_pad)(p3, cnt, g3f, b3v)

    return out[:n]
